# baseline (speedup 1.0000x reference)
import jax
import jax.numpy as jnp
from jax import lax
from jax.experimental import pallas as pl
from jax.experimental.pallas import tpu as pltpu
from jax.experimental.pallas import tpu_sc as plsc

_N, _F, _E = 10000, 128, 320000
_H = _F // 2
_NC, _NS = 2, 16
_C = 128
_EPT = 20480
_EPAD = _NS * _EPT
_CHUNKS = _EPT // _C
_AGG_ROWS = 10240
_ZR = _AGG_ROWS // _NS
_NB = 4


def _sc_body(xu, xv, xy, gu, du, gv, dv, gy, dy, abu, abv, aby,
             ou, ov, oy, *scr):
    ab_f, zeros, agg = scr[0], scr[1], scr[2]
    gs_v = scr[3:3 + _NB]
    dd_v = scr[3 + _NB:3 + 2 * _NB]
    rows = scr[3 + 2 * _NB:3 + 3 * _NB]
    sem_i = scr[3 + 3 * _NB:3 + 4 * _NB]
    sem_g = scr[3 + 4 * _NB:3 + 5 * _NB]
    sem_s = scr[3 + 5 * _NB:3 + 6 * _NB]

    c = lax.axis_index("c")
    s = lax.axis_index("s")
    base = s * _EPT

    def _zr(r, carry):
        for j in range(_H // 16):
            zeros[r, pl.ds(j * 16, 16)] = jnp.zeros((16,), jnp.float32)
        return carry
    lax.fori_loop(0, 128, _zr, 0)

    for (x_h, g_h, d_h, ab_h, o_h) in ((xu, gu, du, abu, ou),
                                       (xv, gv, dv, abv, ov),
                                       (xy, gy, dy, aby, oy)):
        gbase = c * _EPAD + base

        def _issue_idx(g, slot):
            pltpu.async_copy(g_h.at[pl.ds(gbase + g * _C, _C)],
                             gs_v[slot], sem_i[slot])
            pltpu.async_copy(d_h.at[pl.ds(base + g * _C, _C)],
                             dd_v[slot], sem_i[slot])

        def _wait_idx(g, slot):
            pltpu.make_async_copy(g_h.at[pl.ds(gbase + g * _C, _C)],
                                  gs_v[slot], sem_i[slot]).wait()
            pltpu.make_async_copy(d_h.at[pl.ds(base + g * _C, _C)],
                                  dd_v[slot], sem_i[slot]).wait()

        def _issue_gather(slot):
            pltpu.async_copy(x_h.at[gs_v[slot]], rows[slot], sem_g[slot])

        def _wait_gather(slot):
            pltpu.make_async_copy(x_h.at[gs_v[slot]], rows[slot],
                                  sem_g[slot]).wait()

        def _issue_scatter(slot):
            pltpu.async_copy(rows[slot], agg.at[dd_v[slot]], sem_s[slot],
                             add=True)

        def _wait_scatter(slot):
            pltpu.make_async_copy(rows[slot], agg.at[dd_v[slot]],
                                  sem_s[slot]).wait()

        def _compute(slot):
            rv = rows[slot]
            gv_, dv_ = gs_v[slot], dd_v[slot]

            def _gk(k, cc):
                kb = k * 16
                lanes = kb + lax.iota(jnp.int32, 16)
                gidx16 = plsc.load_gather(gv_, [lanes])
                d16 = plsc.load_gather(dv_, [lanes])
                a16 = plsc.load_gather(ab_f, [d16 * 2])
                b16 = plsc.load_gather(ab_f, [gidx16 | 1])
                g16 = 1.0 / (1.0 + jnp.exp(-(a16 + b16)))
                for l in range(16):
                    ge = g16[l]
                    e = kb + l
                    for j in range(_H // 16):
                        sl = pl.ds(j * 16, 16)
                        rv[e, sl] = rv[e, sl] * ge
                return cc
            lax.fori_loop(0, _C // 16, _gk, 0)

        pltpu.sync_copy(ab_h, ab_f.at[pl.ds(0, 2 * _N)])
        ab_f[pl.ds(2 * _N, 16)] = jnp.zeros((16,), jnp.float32)
        ab_f[pl.ds(2 * _N + 16, 16)] = jnp.zeros((16,), jnp.float32)

        zbase = s * _ZR
        for k in range(_ZR // 128):
            pltpu.sync_copy(zeros, agg.at[pl.ds(zbase + k * 128, 128)])
        plsc.subcore_barrier()

        _issue_idx(0, 0)
        _issue_idx(1, 1)
        _issue_idx(2, 2)
        _wait_idx(0, 0)
        _issue_gather(0)
        _wait_idx(1, 1)
        _issue_gather(1)

        def _macro(q, carry):
            for m in range(_NB):
                g = q * _NB + m
                _wait_gather(m)
                _compute(m)
                _issue_scatter(m)

                s3 = (m + 3) % _NB

                @pl.when(g >= 1)
                def _():
                    _wait_scatter(s3)

                @pl.when(g <= _CHUNKS - 4)
                def _():
                    _issue_idx(g + 3, s3)

                s2 = (m + 2) % _NB

                @pl.when(g <= _CHUNKS - 3)
                def _():
                    _wait_idx(g + 2, s2)
                    _issue_gather(s2)
            return carry
        lax.fori_loop(0, _CHUNKS // _NB, _macro, 0)
        _wait_scatter((_CHUNKS - 1) % _NB)
        plsc.subcore_barrier()

        pltpu.sync_copy(agg.at[pl.ds(s * _ZR, _ZR)],
                        o_h.at[c, pl.ds(s * _ZR, _ZR)])
        plsc.subcore_barrier()


_sc_call = pl.kernel(
    _sc_body,
    out_type=(jax.ShapeDtypeStruct((_NC, _AGG_ROWS, _H), jnp.float32),) * 3,
    mesh=plsc.VectorSubcoreMesh(core_axis_name="c", subcore_axis_name="s",
                                num_cores=_NC, num_subcores=_NS),
    scratch_types=(
        [pltpu.VMEM((2 * _N + 32,), jnp.float32),
         pltpu.VMEM((128, _H), jnp.float32),
         pltpu.VMEM_SHARED((_AGG_ROWS, _H), jnp.float32)]
        + [pltpu.VMEM((_C,), jnp.int32)] * _NB
        + [pltpu.VMEM((_C,), jnp.int32)] * _NB
        + [pltpu.VMEM((_C, _H), jnp.float32)] * _NB
        + [pltpu.SemaphoreType.DMA] * (3 * _NB)
    ),
    compiler_params=pltpu.CompilerParams(needs_layout_passes=False,
                                         use_tc_tiling_on_sc=False),
)


_R = 1000


def _ab_body(xu, xv, xy, w_ref, b_ref, o_ref):
    for p, xr in enumerate((xu, xv, xy)):
        o_ref[p] = (jnp.dot(xr[...], w_ref[p],
                            preferred_element_type=jnp.float32) + b_ref[p])


def _ab_call(x_u, x_v, x_y, we2, bias2):
    xspec = pl.BlockSpec((_R, _F), lambda i: (i, 0))
    return pl.pallas_call(
        _ab_body,
        grid=(_N // _R,),
        in_specs=[xspec, xspec, xspec,
                  pl.BlockSpec((3, _F, 2), lambda i: (0, 0, 0)),
                  pl.BlockSpec((3, 1, 2), lambda i: (0, 0, 0))],
        out_specs=pl.BlockSpec((3, _R, 2), lambda i: (0, i, 0)),
        out_shape=jax.ShapeDtypeStruct((3, _N, 2), jnp.float32),
    )(x_u, x_v, x_y, we2, bias2)


def _mlp_body(xu, xv, xy, gu, gv, gy, w1_ref, b1_ref, w2_ref, b2_ref, o_ref):
    for p, (xr, gr) in enumerate(((xu, gu), (xv, gv), (xy, gy))):
        aggr = jnp.concatenate([gr[0], gr[1]], axis=-1)
        w1 = w1_ref[p]
        h = jnp.tanh(jnp.dot(xr[...], w1[:_F],
                             preferred_element_type=jnp.float32)
                     + jnp.dot(aggr, w1[_F:],
                               preferred_element_type=jnp.float32)
                     + b1_ref[p])
        o_ref[p] = jnp.tanh(jnp.dot(h, w2_ref[p],
                                    preferred_element_type=jnp.float32)
                            + b2_ref[p])


def _mlp_call(x_u, x_v, x_y, agg_u, agg_v, agg_y, w1s, b1s, w2s, b2s):
    xspec = pl.BlockSpec((_R, _F), lambda i: (i, 0))
    gspec = pl.BlockSpec((_NC, _R, _H), lambda i: (0, i, 0))
    return pl.pallas_call(
        _mlp_body,
        grid=(_N // _R,),
        in_specs=[xspec, xspec, xspec, gspec, gspec, gspec,
                  pl.BlockSpec((3, 2 * _F, _F), lambda i: (0, 0, 0)),
                  pl.BlockSpec((3, 1, _F), lambda i: (0, 0, 0)),
                  pl.BlockSpec((3, _F, _F), lambda i: (0, 0, 0)),
                  pl.BlockSpec((3, 1, _F), lambda i: (0, 0, 0))],
        out_specs=pl.BlockSpec((3, _R, _F), lambda i: (0, i, 0)),
        out_shape=jax.ShapeDtypeStruct((3, _N, _F), jnp.float32),
    )(x_u, x_v, x_y, agg_u, agg_v, agg_y, w1s, b1s, w2s, b2s)


def kernel(x_u, x_v, x_y, edge_index_u, edge_index_v, edge_index_y,
           We_u, be_u, W1_u, b1_u, W2_u, b2_u,
           We_v, be_v, W1_v, b1_v, W2_v, b2_v,
           We_y, be_y, W1_y, b1_y, W2_y, b2_y):
    we2 = jnp.stack([jnp.stack([w[:_F, 0], w[_F:, 0]], axis=1)
                     for w in (We_u, We_v, We_y)])
    bias2 = jnp.stack([jnp.stack([be[0], jnp.zeros((), jnp.float32)])
                       for be in (be_u, be_v, be_y)]).reshape(3, 1, 2)
    ab = _ab_call(x_u, x_v, x_y, we2, bias2)

    pad = _EPAD - _E
    pad_dst = _N + (jnp.arange(pad, dtype=jnp.int32) % 16)
    pad_src = jnp.zeros((pad,), jnp.int32)

    def _gsplit(ei):
        srcp2 = 2 * jnp.concatenate([ei[0], pad_src])
        return (jnp.concatenate([srcp2, srcp2 + 1]),
                jnp.concatenate([ei[1], pad_dst]))

    def _xsplit(x):
        return x.reshape(2 * _N, _H)

    gu, du = _gsplit(edge_index_u)
    gv, dv = _gsplit(edge_index_v)
    gy, dy = _gsplit(edge_index_y)

    agg_u, agg_v, agg_y = _sc_call(
        _xsplit(x_u), _xsplit(x_v), _xsplit(x_y), gu, du, gv, dv, gy, dy,
        ab[0].reshape(-1), ab[1].reshape(-1), ab[2].reshape(-1))

    w1s = jnp.stack([W1_u, W1_v, W1_y])
    b1s = jnp.stack([b1_u, b1_v, b1_y]).reshape(3, 1, _F)
    w2s = jnp.stack([W2_u, W2_v, W2_y])
    b2s = jnp.stack([b2_u, b2_v, b2_y]).reshape(3, 1, _F)
    return _mlp_call(x_u, x_v, x_y, agg_u, agg_v, agg_y,
                     w1s, b1s, w2s, b2s)


# speedup vs baseline: 1.1848x; 1.1848x over previous
import jax
import jax.numpy as jnp
from jax import lax
from jax.experimental import pallas as pl
from jax.experimental.pallas import tpu as pltpu
from jax.experimental.pallas import tpu_sc as plsc

_N, _F, _E = 10000, 128, 320000
_H = _F // 2
_NC, _NS = 2, 16
_C = 128
_EPT = 20480
_EPAD = _NS * _EPT
_CHUNKS = _EPT // _C
_AGG_ROWS = 10240
_ZR = _AGG_ROWS // _NS
_NB = 4


def _sc_body(xu, xv, xy, gu, du, gv, dv, gy, dy, abu, abv, aby,
             ou, ov, oy, *scr):
    ab_f, zeros, agg = scr[0], scr[1], scr[2]
    gs_v = scr[3:3 + _NB]
    dd_v = scr[3 + _NB:3 + 2 * _NB]
    rows = scr[3 + 2 * _NB:3 + 3 * _NB]
    sem_i = scr[3 + 3 * _NB:3 + 4 * _NB]
    sem_g = scr[3 + 4 * _NB:3 + 5 * _NB]
    sem_s = scr[3 + 5 * _NB:3 + 6 * _NB]

    c = lax.axis_index("c")
    s = lax.axis_index("s")
    base = s * _EPT
    goff = c * _N

    def _zr(r, carry):
        for j in range(_H // 16):
            zeros[r, pl.ds(j * 16, 16)] = jnp.zeros((16,), jnp.float32)
        return carry
    lax.fori_loop(0, 128, _zr, 0)

    for (x_h, g_h, d_h, ab_h, o_h) in ((xu, gu, du, abu, ou),
                                       (xv, gv, dv, abv, ov),
                                       (xy, gy, dy, aby, oy)):
        gbase = c * _EPAD + base

        def _issue_idx(g, slot):
            pltpu.async_copy(g_h.at[pl.ds(gbase + g * _C, _C)],
                             gs_v[slot], sem_i[slot])
            pltpu.async_copy(d_h.at[pl.ds(base + g * _C, _C)],
                             dd_v[slot], sem_i[slot])

        def _wait_idx(g, slot):
            pltpu.make_async_copy(g_h.at[pl.ds(gbase + g * _C, _C)],
                                  gs_v[slot], sem_i[slot]).wait()
            pltpu.make_async_copy(d_h.at[pl.ds(base + g * _C, _C)],
                                  dd_v[slot], sem_i[slot]).wait()

        def _issue_gather(slot):
            pltpu.async_copy(x_h.at[gs_v[slot]], rows[slot], sem_g[slot])

        def _wait_gather(slot):
            pltpu.make_async_copy(x_h.at[gs_v[slot]], rows[slot],
                                  sem_g[slot]).wait()

        def _issue_scatter(slot):
            pltpu.async_copy(rows[slot], agg.at[dd_v[slot]], sem_s[slot],
                             add=True)

        def _wait_scatter(slot):
            pltpu.make_async_copy(rows[slot], agg.at[dd_v[slot]],
                                  sem_s[slot]).wait()

        def _compute(slot):
            rv = rows[slot]
            gv_, dv_ = gs_v[slot], dd_v[slot]

            def _gk(k, cc):
                kb = k * 16
                lanes = kb + lax.iota(jnp.int32, 16)
                gidx16 = plsc.load_gather(gv_, [lanes])
                d16 = plsc.load_gather(dv_, [lanes])
                a16 = plsc.load_gather(ab_f, [d16 * 2])
                b16 = plsc.load_gather(ab_f, [(gidx16 - goff) * 2 + 1])
                g16 = 1.0 / (1.0 + jnp.exp(-(a16 + b16)))
                for l in range(16):
                    ge = g16[l]
                    e = kb + l
                    for j in range(_H // 16):
                        sl = pl.ds(j * 16, 16)
                        rv[e, sl] = rv[e, sl] * ge
                return cc
            lax.fori_loop(0, _C // 16, _gk, 0)

        pltpu.sync_copy(ab_h, ab_f.at[pl.ds(0, 2 * _N)])
        ab_f[pl.ds(2 * _N, 16)] = jnp.zeros((16,), jnp.float32)
        ab_f[pl.ds(2 * _N + 16, 16)] = jnp.zeros((16,), jnp.float32)

        zbase = s * _ZR
        for k in range(_ZR // 128):
            pltpu.sync_copy(zeros, agg.at[pl.ds(zbase + k * 128, 128)])
        plsc.subcore_barrier()

        _issue_idx(0, 0)
        _issue_idx(1, 1)
        _issue_idx(2, 2)
        _wait_idx(0, 0)
        _issue_gather(0)
        _wait_idx(1, 1)
        _issue_gather(1)

        def _macro(q, carry):
            for m in range(_NB):
                g = q * _NB + m
                _wait_gather(m)
                _compute(m)
                _issue_scatter(m)

                s3 = (m + 3) % _NB

                @pl.when(g >= 1)
                def _():
                    _wait_scatter(s3)

                @pl.when(g <= _CHUNKS - 4)
                def _():
                    _issue_idx(g + 3, s3)

                s2 = (m + 2) % _NB

                @pl.when(g <= _CHUNKS - 3)
                def _():
                    _wait_idx(g + 2, s2)
                    _issue_gather(s2)
            return carry
        lax.fori_loop(0, _CHUNKS // _NB, _macro, 0)
        _wait_scatter((_CHUNKS - 1) % _NB)
        plsc.subcore_barrier()

        pltpu.sync_copy(agg.at[pl.ds(s * _ZR, _ZR)],
                        o_h.at[c, pl.ds(s * _ZR, _ZR)])
        plsc.subcore_barrier()


_sc_call = pl.kernel(
    _sc_body,
    out_type=(jax.ShapeDtypeStruct((_NC, _AGG_ROWS, _H), jnp.float32),) * 3,
    mesh=plsc.VectorSubcoreMesh(core_axis_name="c", subcore_axis_name="s",
                                num_cores=_NC, num_subcores=_NS),
    scratch_types=(
        [pltpu.VMEM((2 * _N + 32,), jnp.float32),
         pltpu.VMEM((128, _H), jnp.float32),
         pltpu.VMEM_SHARED((_AGG_ROWS, _H), jnp.float32)]
        + [pltpu.VMEM((_C,), jnp.int32)] * _NB
        + [pltpu.VMEM((_C,), jnp.int32)] * _NB
        + [pltpu.VMEM((_C, _H), jnp.float32)] * _NB
        + [pltpu.SemaphoreType.DMA] * (3 * _NB)
    ),
    compiler_params=pltpu.CompilerParams(needs_layout_passes=False,
                                         use_tc_tiling_on_sc=False),
)


_R = 1000


def _ab_body(xu, xv, xy, w_ref, b_ref, o_ref):
    for p, xr in enumerate((xu, xv, xy)):
        o_ref[p] = (jnp.dot(xr[...], w_ref[p],
                            preferred_element_type=jnp.float32) + b_ref[p])


def _ab_call(x_u, x_v, x_y, we2, bias2):
    xspec = pl.BlockSpec((_R, _F), lambda i: (i, 0))
    return pl.pallas_call(
        _ab_body,
        grid=(_N // _R,),
        in_specs=[xspec, xspec, xspec,
                  pl.BlockSpec((3, _F, 2), lambda i: (0, 0, 0)),
                  pl.BlockSpec((3, 1, 2), lambda i: (0, 0, 0))],
        out_specs=pl.BlockSpec((3, _R, 2), lambda i: (0, i, 0)),
        out_shape=jax.ShapeDtypeStruct((3, _N, 2), jnp.float32),
    )(x_u, x_v, x_y, we2, bias2)


def _mlp_body(xu, xv, xy, gu, gv, gy, w1_ref, b1_ref, w2_ref, b2_ref, o_ref):
    for p, (xr, gr) in enumerate(((xu, gu), (xv, gv), (xy, gy))):
        aggr = jnp.concatenate([gr[0], gr[1]], axis=-1)
        w1 = w1_ref[p]
        h = jnp.tanh(jnp.dot(xr[...], w1[:_F],
                             preferred_element_type=jnp.float32)
                     + jnp.dot(aggr, w1[_F:],
                               preferred_element_type=jnp.float32)
                     + b1_ref[p])
        o_ref[p] = jnp.tanh(jnp.dot(h, w2_ref[p],
                                    preferred_element_type=jnp.float32)
                            + b2_ref[p])


def _mlp_call(x_u, x_v, x_y, agg_u, agg_v, agg_y, w1s, b1s, w2s, b2s):
    xspec = pl.BlockSpec((_R, _F), lambda i: (i, 0))
    gspec = pl.BlockSpec((_NC, _R, _H), lambda i: (0, i, 0))
    return pl.pallas_call(
        _mlp_body,
        grid=(_N // _R,),
        in_specs=[xspec, xspec, xspec, gspec, gspec, gspec,
                  pl.BlockSpec((3, 2 * _F, _F), lambda i: (0, 0, 0)),
                  pl.BlockSpec((3, 1, _F), lambda i: (0, 0, 0)),
                  pl.BlockSpec((3, _F, _F), lambda i: (0, 0, 0)),
                  pl.BlockSpec((3, 1, _F), lambda i: (0, 0, 0))],
        out_specs=pl.BlockSpec((3, _R, _F), lambda i: (0, i, 0)),
        out_shape=jax.ShapeDtypeStruct((3, _N, _F), jnp.float32),
    )(x_u, x_v, x_y, agg_u, agg_v, agg_y, w1s, b1s, w2s, b2s)


def kernel(x_u, x_v, x_y, edge_index_u, edge_index_v, edge_index_y,
           We_u, be_u, W1_u, b1_u, W2_u, b2_u,
           We_v, be_v, W1_v, b1_v, W2_v, b2_v,
           We_y, be_y, W1_y, b1_y, W2_y, b2_y):
    we2 = jnp.stack([jnp.stack([w[:_F, 0], w[_F:, 0]], axis=1)
                     for w in (We_u, We_v, We_y)])
    bias2 = jnp.stack([jnp.stack([be[0], jnp.zeros((), jnp.float32)])
                       for be in (be_u, be_v, be_y)]).reshape(3, 1, 2)
    ab = _ab_call(x_u, x_v, x_y, we2, bias2)

    pad = _EPAD - _E
    pad_dst = _N + (jnp.arange(pad, dtype=jnp.int32) % 16)
    pad_src = jnp.zeros((pad,), jnp.int32)

    def _gsplit(ei):
        srcp = jnp.concatenate([ei[0], pad_src])
        return (jnp.concatenate([srcp, srcp + _N]),
                jnp.concatenate([ei[1], pad_dst]))

    def _xsplit(x):
        return jnp.concatenate([x[:, :_H], x[:, _H:]], axis=0)

    gu, du = _gsplit(edge_index_u)
    gv, dv = _gsplit(edge_index_v)
    gy, dy = _gsplit(edge_index_y)

    agg_u, agg_v, agg_y = _sc_call(
        _xsplit(x_u), _xsplit(x_v), _xsplit(x_y), gu, du, gv, dv, gy, dy,
        ab[0].reshape(-1), ab[1].reshape(-1), ab[2].reshape(-1))

    w1s = jnp.stack([W1_u, W1_v, W1_y])
    b1s = jnp.stack([b1_u, b1_v, b1_y]).reshape(3, 1, _F)
    w2s = jnp.stack([W2_u, W2_v, W2_y])
    b2s = jnp.stack([b2_u, b2_v, b2_y]).reshape(3, 1, _F)
    return _mlp_call(x_u, x_v, x_y, agg_u, agg_v, agg_y,
                     w1s, b1s, w2s, b2s)


# speedup vs baseline: 1.2207x; 1.0303x over previous
import jax
import jax.numpy as jnp
from jax import lax
from jax.experimental import pallas as pl
from jax.experimental.pallas import tpu as pltpu
from jax.experimental.pallas import tpu_sc as plsc

_N, _F, _E = 10000, 128, 320000
_H = _F // 2
_NC, _NS = 2, 16
_C = 128
_EPT = 20480
_EPAD = _NS * _EPT
_CHUNKS = _EPT // _C
_AGG_ROWS = 10240
_ZR = _AGG_ROWS // _NS
_NB = 5


def _sc_body(xu, xv, xy, gu, du, gv, dv, gy, dy, abu, abv, aby,
             ou, ov, oy, *scr):
    ab_f, zeros, agg = scr[0], scr[1], scr[2]
    gs_v = scr[3:3 + _NB]
    dd_v = scr[3 + _NB:3 + 2 * _NB]
    rows = scr[3 + 2 * _NB:3 + 3 * _NB]
    sem_i = scr[3 + 3 * _NB:3 + 4 * _NB]
    sem_g = scr[3 + 4 * _NB:3 + 5 * _NB]
    sem_s = scr[3 + 5 * _NB:3 + 6 * _NB]

    c = lax.axis_index("c")
    s = lax.axis_index("s")
    base = s * _EPT
    goff = c * _N

    def _zr(r, carry):
        for j in range(_H // 16):
            zeros[r, pl.ds(j * 16, 16)] = jnp.zeros((16,), jnp.float32)
        return carry
    lax.fori_loop(0, 128, _zr, 0)

    for (x_h, g_h, d_h, ab_h, o_h) in ((xu, gu, du, abu, ou),
                                       (xv, gv, dv, abv, ov),
                                       (xy, gy, dy, aby, oy)):
        gbase = c * _EPAD + base

        def _issue_idx(g, slot):
            pltpu.async_copy(g_h.at[pl.ds(gbase + g * _C, _C)],
                             gs_v[slot], sem_i[slot])
            pltpu.async_copy(d_h.at[pl.ds(base + g * _C, _C)],
                             dd_v[slot], sem_i[slot])

        def _wait_idx(g, slot):
            pltpu.make_async_copy(g_h.at[pl.ds(gbase + g * _C, _C)],
                                  gs_v[slot], sem_i[slot]).wait()
            pltpu.make_async_copy(d_h.at[pl.ds(base + g * _C, _C)],
                                  dd_v[slot], sem_i[slot]).wait()

        def _issue_gather(slot):
            pltpu.async_copy(x_h.at[gs_v[slot]], rows[slot], sem_g[slot])

        def _wait_gather(slot):
            pltpu.make_async_copy(x_h.at[gs_v[slot]], rows[slot],
                                  sem_g[slot]).wait()

        def _issue_scatter(slot):
            pltpu.async_copy(rows[slot], agg.at[dd_v[slot]], sem_s[slot],
                             add=True)

        def _wait_scatter(slot):
            pltpu.make_async_copy(rows[slot], agg.at[dd_v[slot]],
                                  sem_s[slot]).wait()

        def _compute(slot):
            rv = rows[slot]
            gv_, dv_ = gs_v[slot], dd_v[slot]

            def _gk(k, cc):
                kb = k * 16
                lanes = kb + lax.iota(jnp.int32, 16)
                gidx16 = plsc.load_gather(gv_, [lanes])
                d16 = plsc.load_gather(dv_, [lanes])
                a16 = plsc.load_gather(ab_f, [d16 * 2])
                b16 = plsc.load_gather(ab_f, [(gidx16 - goff) * 2 + 1])
                g16 = 1.0 / (1.0 + jnp.exp(-(a16 + b16)))
                for l in range(16):
                    ge = g16[l]
                    e = kb + l
                    for j in range(_H // 16):
                        sl = pl.ds(j * 16, 16)
                        rv[e, sl] = rv[e, sl] * ge
                return cc
            lax.fori_loop(0, _C // 16, _gk, 0)

        pltpu.sync_copy(ab_h, ab_f.at[pl.ds(0, 2 * _N)])
        ab_f[pl.ds(2 * _N, 16)] = jnp.zeros((16,), jnp.float32)
        ab_f[pl.ds(2 * _N + 16, 16)] = jnp.zeros((16,), jnp.float32)

        zbase = s * _ZR
        for k in range(_ZR // 128):
            pltpu.sync_copy(zeros, agg.at[pl.ds(zbase + k * 128, 128)])
        plsc.subcore_barrier()

        for k in range(4):
            _issue_idx(k, k)
        for k in range(3):
            _wait_idx(k, k)
            _issue_gather(k)

        def _macro(q, carry):
            for m in range(_NB):
                g = q * _NB + m
                _wait_gather(m)
                _compute(m)
                _issue_scatter(m)

                s4 = (m + 4) % _NB

                @pl.when(g >= 1)
                def _():
                    _wait_scatter(s4)

                @pl.when(g <= _CHUNKS - 5)
                def _():
                    _issue_idx(g + 4, s4)

                s3 = (m + 3) % _NB

                @pl.when(g <= _CHUNKS - 4)
                def _():
                    _wait_idx(g + 3, s3)
                    _issue_gather(s3)
            return carry
        lax.fori_loop(0, _CHUNKS // _NB, _macro, 0)
        _wait_scatter((_CHUNKS - 1) % _NB)
        plsc.subcore_barrier()

        pltpu.sync_copy(agg.at[pl.ds(s * _ZR, _ZR)],
                        o_h.at[c, pl.ds(s * _ZR, _ZR)])
        plsc.subcore_barrier()


_sc_call = pl.kernel(
    _sc_body,
    out_type=(jax.ShapeDtypeStruct((_NC, _AGG_ROWS, _H), jnp.float32),) * 3,
    mesh=plsc.VectorSubcoreMesh(core_axis_name="c", subcore_axis_name="s",
                                num_cores=_NC, num_subcores=_NS),
    scratch_types=(
        [pltpu.VMEM((2 * _N + 32,), jnp.float32),
         pltpu.VMEM((128, _H), jnp.float32),
         pltpu.VMEM_SHARED((_AGG_ROWS, _H), jnp.float32)]
        + [pltpu.VMEM((_C,), jnp.int32)] * _NB
        + [pltpu.VMEM((_C,), jnp.int32)] * _NB
        + [pltpu.VMEM((_C, _H), jnp.float32)] * _NB
        + [pltpu.SemaphoreType.DMA] * (3 * _NB)
    ),
    compiler_params=pltpu.CompilerParams(needs_layout_passes=False,
                                         use_tc_tiling_on_sc=False),
)


_R = 1000


def _ab_body(xu, xv, xy, w_ref, b_ref, o_ref):
    for p, xr in enumerate((xu, xv, xy)):
        o_ref[p] = (jnp.dot(xr[...], w_ref[p],
                            preferred_element_type=jnp.float32) + b_ref[p])


def _ab_call(x_u, x_v, x_y, we2, bias2):
    xspec = pl.BlockSpec((_R, _F), lambda i: (i, 0))
    return pl.pallas_call(
        _ab_body,
        grid=(_N // _R,),
        in_specs=[xspec, xspec, xspec,
                  pl.BlockSpec((3, _F, 2), lambda i: (0, 0, 0)),
                  pl.BlockSpec((3, 1, 2), lambda i: (0, 0, 0))],
        out_specs=pl.BlockSpec((3, _R, 2), lambda i: (0, i, 0)),
        out_shape=jax.ShapeDtypeStruct((3, _N, 2), jnp.float32),
    )(x_u, x_v, x_y, we2, bias2)


def _mlp_body(xu, xv, xy, gu, gv, gy, w1_ref, b1_ref, w2_ref, b2_ref, o_ref):
    for p, (xr, gr) in enumerate(((xu, gu), (xv, gv), (xy, gy))):
        aggr = jnp.concatenate([gr[0], gr[1]], axis=-1)
        w1 = w1_ref[p]
        h = jnp.tanh(jnp.dot(xr[...], w1[:_F],
                             preferred_element_type=jnp.float32)
                     + jnp.dot(aggr, w1[_F:],
                               preferred_element_type=jnp.float32)
                     + b1_ref[p])
        o_ref[p] = jnp.tanh(jnp.dot(h, w2_ref[p],
                                    preferred_element_type=jnp.float32)
                            + b2_ref[p])


def _mlp_call(x_u, x_v, x_y, agg_u, agg_v, agg_y, w1s, b1s, w2s, b2s):
    xspec = pl.BlockSpec((_R, _F), lambda i: (i, 0))
    gspec = pl.BlockSpec((_NC, _R, _H), lambda i: (0, i, 0))
    return pl.pallas_call(
        _mlp_body,
        grid=(_N // _R,),
        in_specs=[xspec, xspec, xspec, gspec, gspec, gspec,
                  pl.BlockSpec((3, 2 * _F, _F), lambda i: (0, 0, 0)),
                  pl.BlockSpec((3, 1, _F), lambda i: (0, 0, 0)),
                  pl.BlockSpec((3, _F, _F), lambda i: (0, 0, 0)),
                  pl.BlockSpec((3, 1, _F), lambda i: (0, 0, 0))],
        out_specs=pl.BlockSpec((3, _R, _F), lambda i: (0, i, 0)),
        out_shape=jax.ShapeDtypeStruct((3, _N, _F), jnp.float32),
    )(x_u, x_v, x_y, agg_u, agg_v, agg_y, w1s, b1s, w2s, b2s)


def kernel(x_u, x_v, x_y, edge_index_u, edge_index_v, edge_index_y,
           We_u, be_u, W1_u, b1_u, W2_u, b2_u,
           We_v, be_v, W1_v, b1_v, W2_v, b2_v,
           We_y, be_y, W1_y, b1_y, W2_y, b2_y):
    we2 = jnp.stack([jnp.stack([w[:_F, 0], w[_F:, 0]], axis=1)
                     for w in (We_u, We_v, We_y)])
    bias2 = jnp.stack([jnp.stack([be[0], jnp.zeros((), jnp.float32)])
                       for be in (be_u, be_v, be_y)]).reshape(3, 1, 2)
    ab = _ab_call(x_u, x_v, x_y, we2, bias2)

    pad = _EPAD - _E
    pad_dst = _N + (jnp.arange(pad, dtype=jnp.int32) % 16)
    pad_src = jnp.zeros((pad,), jnp.int32)

    def _gsplit(ei):
        srcp = jnp.concatenate([ei[0], pad_src])
        return (jnp.concatenate([srcp, srcp + _N]),
                jnp.concatenate([ei[1], pad_dst]))

    def _xsplit(x):
        return jnp.concatenate([x[:, :_H], x[:, _H:]], axis=0)

    gu, du = _gsplit(edge_index_u)
    gv, dv = _gsplit(edge_index_v)
    gy, dy = _gsplit(edge_index_y)

    agg_u, agg_v, agg_y = _sc_call(
        _xsplit(x_u), _xsplit(x_v), _xsplit(x_y), gu, du, gv, dv, gy, dy,
        ab[0].reshape(-1), ab[1].reshape(-1), ab[2].reshape(-1))

    w1s = jnp.stack([W1_u, W1_v, W1_y])
    b1s = jnp.stack([b1_u, b1_v, b1_y]).reshape(3, 1, _F)
    w2s = jnp.stack([W2_u, W2_v, W2_y])
    b2s = jnp.stack([b2_u, b2_v, b2_y]).reshape(3, 1, _F)
    return _mlp_call(x_u, x_v, x_y, agg_u, agg_v, agg_y,
                     w1s, b1s, w2s, b2s)
